# baseline (speedup 1.0000x reference)
import functools

import jax
import jax.numpy as jnp
from jax import lax
from jax.experimental import pallas as pl
from jax.experimental.pallas import tpu as pltpu
from jax.experimental.pallas import tpu_sc as plsc

B, L, D = 4096, 50, 64
LP = 64
NW = 32
RPW = B // NW
K = 8
NPH = RPW // K


@jax.jit
def _poi_gather(idx, table):
    mesh = plsc.VectorSubcoreMesh(core_axis_name="c", subcore_axis_name="s")

    @functools.partial(
        pl.kernel,
        out_type=jax.ShapeDtypeStruct((B, 56, 128), jnp.float32),
        mesh=mesh,
        compiler_params=pltpu.CompilerParams(use_tc_tiling_on_sc=False),
        scratch_types=[
            pltpu.VMEM((RPW * LP,), jnp.int32),
            pltpu.VMEM((2, K, L, D), jnp.float32),
            pltpu.SemaphoreType.DMA,
            pltpu.SemaphoreType.DMA,
        ],
    )
    def k(idx_hbm, table_hbm, out_hbm, idx_v, rows_v, g_sem, o_sem):
        wid = lax.axis_index("s") * 2 + lax.axis_index("c")
        base = wid * RPW
        pltpu.sync_copy(idx_hbm.at[pl.ds(base * LP, RPW * LP)], idx_v)

        def fire(p, bank):
            for b in range(K):
                pltpu.async_copy(
                    table_hbm.at[idx_v.at[pl.ds((p * K + b) * LP, L)]],
                    rows_v.at[bank, b], g_sem)

        def wait_gathers():
            for _ in range(K):
                pltpu.make_async_copy(
                    table_hbm.at[idx_v.at[pl.ds(0, L)]], rows_v.at[0, 0],
                    g_sem).wait()

        def puts(p, bank):
            for b in range(K):
                pltpu.async_copy(
                    rows_v.at[bank, b],
                    out_hbm.at[base + p * K + b, pl.ds(0, L), pl.ds(0, D)],
                    o_sem)

        def wait_puts():
            for _ in range(K):
                pltpu.make_async_copy(
                    rows_v.at[0, 0],
                    out_hbm.at[0, pl.ds(0, L), pl.ds(0, D)], o_sem).wait()

        fire(0, 0)

        def body(i, carry):
            for q in range(2):
                p = 2 * i + q

                @pl.when(p > 0)
                def _drain():
                    wait_puts()

                @pl.when(p < NPH - 1)
                def _prefetch():
                    fire(p + 1, 1 - q)

                wait_gathers()
                puts(p, q)
            return carry

        lax.fori_loop(0, NPH // 2, body, 0)
        wait_puts()

    return k(idx, table)


def kernel(x, POI):
    xp = jnp.pad(x.astype(jnp.int32), ((0, 0), (0, LP - L)))
    big = _poi_gather(xp.reshape(B * LP), POI)
    return big[:, :L, :D]
